# Initial kernel scaffold; baseline (speedup 1.0000x reference)
import jax
import jax.numpy as jnp
from jax.experimental import pallas as pl


def kernel(atype, table):
    raise NotImplementedError("write your pallas kernel here")


# speedup vs baseline: 3.4274x; 3.4274x over previous
import functools

import jax
import jax.numpy as jnp
from jax import lax
from jax.experimental import pallas as pl
from jax.experimental.pallas import tpu as pltpu
from jax.experimental.pallas import tpu_sc as plsc

_CHUNK = 128


@functools.lru_cache(maxsize=None)
def _make_gather(n_rows: int, n_chunks_total: int, embed_dim: int):
    info = plsc.get_sparse_core_info()
    nw = info.num_cores * info.num_subcores
    assert n_chunks_total % nw == 0
    chunks_per_w = n_chunks_total // nw

    mesh = plsc.VectorSubcoreMesh(core_axis_name="c", subcore_axis_name="s")

    @functools.partial(
        pl.kernel,
        mesh=mesh,
        out_type=jax.ShapeDtypeStruct(
            (n_chunks_total * _CHUNK, embed_dim), jnp.float32
        ),
        scratch_types=[
            pltpu.VMEM((chunks_per_w, _CHUNK), jnp.int32),
            pltpu.VMEM((_CHUNK, embed_dim), jnp.float32),
            pltpu.SemaphoreType.DMA,
        ],
        compiler_params=pltpu.CompilerParams(use_tc_tiling_on_sc=False),
    )
    def k(table_hbm, idx_hbm, out_hbm, idx_v, rows_v, sem):
        wid = lax.axis_index("s") * info.num_cores + lax.axis_index("c")
        c_base = wid * chunks_per_w
        pltpu.sync_copy(idx_hbm.at[pl.ds(c_base, chunks_per_w)], idx_v)

        def body(j, _):
            pltpu.async_copy(table_hbm.at[idx_v.at[j]], rows_v, sem).wait()
            pltpu.sync_copy(
                rows_v,
                out_hbm.at[pl.ds((c_base + j) * _CHUNK, _CHUNK)],
            )
            return 0

        lax.fori_loop(0, chunks_per_w, body, 0)

    return k


def kernel(atype, table):
    b0, b1 = atype.shape
    n_rows, embed_dim = table.shape
    total = b0 * b1
    assert total % _CHUNK == 0
    idx2d = atype.reshape(total // _CHUNK, _CHUNK).astype(jnp.int32)
    out = _make_gather(n_rows, total // _CHUNK, embed_dim)(table, idx2d)
    return out.reshape(b0, b1, embed_dim)


# speedup vs baseline: 3.5634x; 1.0397x over previous
import functools

import jax
import jax.numpy as jnp
from jax import lax
from jax.experimental import pallas as pl
from jax.experimental.pallas import tpu as pltpu
from jax.experimental.pallas import tpu_sc as plsc

_CHUNK = 128
_GRP = 4


@functools.lru_cache(maxsize=None)
def _make_gather(n_rows: int, n_chunks_total: int, embed_dim: int):
    info = plsc.get_sparse_core_info()
    nw = info.num_cores * info.num_subcores
    assert n_chunks_total % (nw * 2 * _GRP) == 0
    chunks_per_w = n_chunks_total // nw
    n_groups = chunks_per_w // _GRP

    mesh = plsc.VectorSubcoreMesh(core_axis_name="c", subcore_axis_name="s")

    @functools.partial(
        pl.kernel,
        mesh=mesh,
        out_type=jax.ShapeDtypeStruct(
            (n_chunks_total * _CHUNK, embed_dim), jnp.float32
        ),
        scratch_types=[
            pltpu.VMEM((chunks_per_w, _CHUNK), jnp.int32),
            pltpu.VMEM((_GRP * _CHUNK, embed_dim), jnp.float32),
            pltpu.VMEM((_GRP * _CHUNK, embed_dim), jnp.float32),
            pltpu.SemaphoreType.DMA,
            pltpu.SemaphoreType.DMA,
            pltpu.SemaphoreType.DMA,
            pltpu.SemaphoreType.DMA,
        ],
        compiler_params=pltpu.CompilerParams(use_tc_tiling_on_sc=False),
    )
    def k(table_hbm, idx_hbm, out_hbm, idx_v, rows0, rows1, g0, g1, s0, s1):
        sets = ((rows0, g0, s0), (rows1, g1, s1))
        wid = lax.axis_index("s") * info.num_cores + lax.axis_index("c")
        c_base = wid * chunks_per_w
        pltpu.sync_copy(idx_hbm.at[pl.ds(c_base, chunks_per_w)], idx_v)

        def gather(g, p):
            rows, sg, _ = sets[p]
            return [
                pltpu.make_async_copy(
                    table_hbm.at[idx_v.at[g * _GRP + b]],
                    rows.at[pl.ds(b * _CHUNK, _CHUNK)],
                    sg,
                )
                for b in range(_GRP)
            ]

        def scatter(g, p):
            rows, _, sc = sets[p]
            return pltpu.make_async_copy(
                rows,
                out_hbm.at[pl.ds((c_base + g * _GRP) * _CHUNK, _GRP * _CHUNK)],
                sc,
            )

        def gather_start(g, p):
            for c in gather(g, p):
                c.start()

        def gather_wait(g, p):
            for c in gather(g, p):
                c.wait()

        gather_start(0, 0)
        gather_start(1, 1)
        gather_wait(0, 0)
        scatter(0, 0).start()
        gather_wait(1, 1)
        scatter(1, 1).start()

        def body(i, _):
            ga = 2 * i
            scatter(ga - 2, 0).wait()
            gather_start(ga, 0)
            scatter(ga - 1, 1).wait()
            gather_start(ga + 1, 1)
            gather_wait(ga, 0)
            scatter(ga, 0).start()
            gather_wait(ga + 1, 1)
            scatter(ga + 1, 1).start()
            return 0

        lax.fori_loop(1, n_groups // 2, body, 0)
        scatter(n_groups - 2, 0).wait()
        scatter(n_groups - 1, 1).wait()

    return k


def kernel(atype, table):
    b0, b1 = atype.shape
    n_rows, embed_dim = table.shape
    total = b0 * b1
    assert total % _CHUNK == 0
    idx2d = atype.reshape(total // _CHUNK, _CHUNK).astype(jnp.int32)
    out = _make_gather(n_rows, total // _CHUNK, embed_dim)(table, idx2d)
    return out.reshape(b0, b1, embed_dim)


# speedup vs baseline: 3.5817x; 1.0051x over previous
import functools

import jax
import jax.numpy as jnp
from jax import lax
from jax.experimental import pallas as pl
from jax.experimental.pallas import tpu as pltpu
from jax.experimental.pallas import tpu_sc as plsc

_MAXG = 128


@functools.lru_cache(maxsize=None)
def _make_lookup(n_rows: int, n_atoms: int, n_per_atom: int, embed_dim: int):
    info = plsc.get_sparse_core_info()
    nw = info.num_cores * info.num_subcores
    assert n_atoms % (2 * nw) == 0
    atoms_per_w = n_atoms // nw
    segs = []
    off = 0
    while off < n_per_atom:
        n = min(_MAXG, n_per_atom - off)
        segs.append((off, n))
        off += n

    mesh = plsc.VectorSubcoreMesh(core_axis_name="c", subcore_axis_name="s")

    @functools.partial(
        pl.kernel,
        mesh=mesh,
        out_type=jax.ShapeDtypeStruct(
            (n_atoms, n_per_atom, embed_dim), jnp.float32
        ),
        scratch_types=[
            pltpu.VMEM((atoms_per_w, n_per_atom), jnp.int32),
            pltpu.VMEM((n_per_atom, embed_dim), jnp.float32),
            pltpu.VMEM((n_per_atom, embed_dim), jnp.float32),
            pltpu.SemaphoreType.DMA,
            pltpu.SemaphoreType.DMA,
            pltpu.SemaphoreType.DMA,
            pltpu.SemaphoreType.DMA,
        ],
        compiler_params=pltpu.CompilerParams(use_tc_tiling_on_sc=False),
    )
    def k(table_hbm, idx_hbm, out_hbm, idx_v, st0, st1, g0, g1, s0, s1):
        bufs = ((st0, g0, s0), (st1, g1, s1))
        wid = lax.axis_index("s") * info.num_cores + lax.axis_index("c")
        a_base = wid * atoms_per_w
        pltpu.sync_copy(idx_hbm.at[pl.ds(a_base, atoms_per_w)], idx_v)

        def gathers(a, p):
            st, sg, _ = bufs[p]
            return [
                pltpu.make_async_copy(
                    table_hbm.at[idx_v.at[a, pl.ds(off, n)]],
                    st.at[pl.ds(off, n)],
                    sg,
                )
                for off, n in segs
            ]

        def scat(a, p):
            st, _, sc = bufs[p]
            return pltpu.make_async_copy(st, out_hbm.at[a_base + a], sc)

        for c in gathers(0, 0):
            c.start()
        for c in gathers(1, 1):
            c.start()
        for c in gathers(0, 0):
            c.wait()
        scat(0, 0).start()
        for c in gathers(1, 1):
            c.wait()
        scat(1, 1).start()

        def body(i, _):
            a0 = 2 * i
            scat(a0 - 2, 0).wait()
            for c in gathers(a0, 0):
                c.start()
            scat(a0 - 1, 1).wait()
            for c in gathers(a0 + 1, 1):
                c.start()
            for c in gathers(a0, 0):
                c.wait()
            scat(a0, 0).start()
            for c in gathers(a0 + 1, 1):
                c.wait()
            scat(a0 + 1, 1).start()
            return 0

        lax.fori_loop(1, atoms_per_w // 2, body, 0)
        scat(atoms_per_w - 2, 0).wait()
        scat(atoms_per_w - 1, 1).wait()

    return k


def kernel(atype, table):
    b0, b1 = atype.shape
    n_rows, embed_dim = table.shape
    idx = atype.astype(jnp.int32)
    return _make_lookup(n_rows, b0, b1, embed_dim)(table, idx)


# speedup vs baseline: 4.6416x; 1.2959x over previous
import functools

import jax
import jax.numpy as jnp
from jax import lax
from jax.experimental import pallas as pl
from jax.experimental.pallas import tpu as pltpu
from jax.experimental.pallas import tpu_sc as plsc

_MAXG = 128


@functools.lru_cache(maxsize=None)
def _make_lookup(n_rows: int, n_atoms: int, n_per_atom: int, embed_dim: int):
    info = plsc.get_sparse_core_info()
    nw = info.num_cores * info.num_subcores
    assert n_atoms % (2 * nw) == 0
    atoms_per_w = n_atoms // nw
    segs = []
    off = 0
    while off < n_per_atom:
        n = min(_MAXG, n_per_atom - off)
        segs.append((off, n))
        off += n

    mesh = plsc.VectorSubcoreMesh(core_axis_name="c", subcore_axis_name="s")

    @functools.partial(
        pl.kernel,
        mesh=mesh,
        out_type=jax.ShapeDtypeStruct(
            (n_atoms, n_per_atom, embed_dim), jnp.float32
        ),
        scratch_types=[
            pltpu.VMEM((atoms_per_w, n_per_atom), jnp.int32),
            pltpu.VMEM((n_per_atom, embed_dim), jnp.float32),
            pltpu.VMEM((n_per_atom, embed_dim), jnp.float32),
            pltpu.VMEM_SHARED((n_rows, embed_dim), jnp.float32),
            pltpu.SemaphoreType.DMA,
            pltpu.SemaphoreType.DMA,
            pltpu.SemaphoreType.DMA,
            pltpu.SemaphoreType.DMA,
        ],
        compiler_params=pltpu.CompilerParams(use_tc_tiling_on_sc=False),
    )
    def k(table_hbm, idx_hbm, out_hbm, idx_v, st0, st1, table_sp, g0, g1, s0, s1):
        bufs = ((st0, g0, s0), (st1, g1, s1))
        sid = lax.axis_index("s")
        wid = sid * info.num_cores + lax.axis_index("c")
        a_base = wid * atoms_per_w

        @pl.when(sid == 0)
        def _():
            pltpu.sync_copy(table_hbm, table_sp)

        pltpu.sync_copy(idx_hbm.at[pl.ds(a_base, atoms_per_w)], idx_v)
        plsc.subcore_barrier()

        def gathers(a, p):
            st, sg, _ = bufs[p]
            return [
                pltpu.make_async_copy(
                    table_sp.at[idx_v.at[a, pl.ds(off, n)]],
                    st.at[pl.ds(off, n)],
                    sg,
                )
                for off, n in segs
            ]

        def scat(a, p):
            st, _, sc = bufs[p]
            return pltpu.make_async_copy(st, out_hbm.at[a_base + a], sc)

        for c in gathers(0, 0):
            c.start()
        for c in gathers(1, 1):
            c.start()
        for c in gathers(0, 0):
            c.wait()
        scat(0, 0).start()
        for c in gathers(1, 1):
            c.wait()
        scat(1, 1).start()

        def body(i, _):
            a0 = 2 * i
            scat(a0 - 2, 0).wait()
            for c in gathers(a0, 0):
                c.start()
            scat(a0 - 1, 1).wait()
            for c in gathers(a0 + 1, 1):
                c.start()
            for c in gathers(a0, 0):
                c.wait()
            scat(a0, 0).start()
            for c in gathers(a0 + 1, 1):
                c.wait()
            scat(a0 + 1, 1).start()
            return 0

        lax.fori_loop(1, atoms_per_w // 2, body, 0)
        scat(atoms_per_w - 2, 0).wait()
        scat(atoms_per_w - 1, 1).wait()

    return k


def kernel(atype, table):
    b0, b1 = atype.shape
    n_rows, embed_dim = table.shape
    idx = atype.astype(jnp.int32)
    return _make_lookup(n_rows, b0, b1, embed_dim)(table, idx)
